# baseline (speedup 1.0000x reference)
import functools

import jax
import jax.numpy as jnp
from jax.experimental import pallas as pl
from jax.experimental.pallas import tpu as pltpu

N = 4096
F_IN = 128
F_OUT = 64
NHEADS = 4
ALPHA = 0.2

BR = 256
BC = 1024
HW = 128


def _prep_kernel(h_ref, w_ref, a1_ref, a2_ref,
                 wh_ref, s1_ref, s2t_ref, s2max_ref, runmax_ref):
    i = pl.program_id(0)
    ni = pl.num_programs(0)
    nheads = s1_ref.shape[1]
    fout = w_ref.shape[1] // nheads

    wh = jnp.dot(h_ref[:], w_ref[:], preferred_element_type=jnp.float32)
    s1 = jnp.dot(wh, a1_ref[:], preferred_element_type=jnp.float32)
    s2 = jnp.dot(wh, a2_ref[:], preferred_element_type=jnp.float32)
    s1_ref[:] = s1

    br = wh.shape[0]
    ones = jnp.ones((br, 1), jnp.float32)
    zpad = jnp.zeros((br, HW - fout - 1), jnp.float32)
    parts = []
    for k in range(nheads):
        parts += [wh[:, k * fout:(k + 1) * fout], ones, zpad]
    wh_ref[:] = jnp.concatenate(parts, axis=1)

    s2t = jnp.concatenate(
        [s2.T, jnp.zeros((8 - nheads, br), jnp.float32)], axis=0)
    s2t_ref[:] = s2t

    bmax = jnp.max(s2, axis=0, keepdims=True)

    @pl.when(i == 0)
    def _():
        runmax_ref[:] = jnp.full_like(runmax_ref, -jnp.inf)

    runmax_ref[:] = jnp.maximum(runmax_ref[:], bmax)

    @pl.when(i == ni - 1)
    def _():
        s2max_ref[:] = runmax_ref[:]


def _flash_kernel(nheads, fout, out_transform,
                  adj_ref, s1_ref, s2t_ref, s2max_ref, wh_ref,
                  out_ref, acc_ref, row_ref):
    j = pl.program_id(1)
    nj = pl.num_programs(1)

    @pl.when(j == 0)
    def _init():
        acc_ref[:] = jnp.zeros_like(acc_ref)
        for k in range(nheads):
            s1c = s1_ref[:, k:k + 1]
            s2m = s2max_ref[0:1, k:k + 1]
            t = s1c + s2m
            mrow = jnp.maximum(jnp.where(t >= 0.0, t, ALPHA * t), 0.0)
            row_ref[:, k:k + 1] = jnp.exp(t - mrow)
            row_ref[:, nheads + k:nheads + k + 1] = jnp.exp(
                ALPHA * t - mrow)
            row_ref[:, 2 * nheads + k:2 * nheads + k + 1] = jnp.exp(-mrow)
            row_ref[:, 3 * nheads + k:3 * nheads + k + 1] = -s1c

    mask = adj_ref[:] > 0.0

    for k in range(nheads):
        s2r = s2t_ref[k:k + 1, :]
        s2m = s2max_ref[0:1, k:k + 1]
        e1 = row_ref[:, k:k + 1]
        f1 = row_ref[:, nheads + k:nheads + k + 1]
        g = row_ref[:, 2 * nheads + k:2 * nheads + k + 1]
        ns1 = row_ref[:, 3 * nheads + k:3 * nheads + k + 1]
        e2 = jnp.exp(s2r - s2m)
        f2 = jnp.exp(ALPHA * (s2r - s2m))

        c = s2r >= ns1
        u = jnp.where(c, e2, f2)
        v = jnp.where(c, e1, f1)
        p = jnp.where(mask, u * v, g)
        whb = wh_ref[pl.ds(j * BC, BC), k * HW:(k + 1) * HW]
        acc_ref[:, k * HW:(k + 1) * HW] += jnp.dot(
            p, whb, preferred_element_type=jnp.float32)

    @pl.when(j == nj - 1)
    def _finish():
        for k in range(nheads):
            hp = (acc_ref[:, k * HW:k * HW + fout]
                  / acc_ref[:, k * HW + fout:k * HW + fout + 1])
            out_ref[:, k * fout:(k + 1) * fout] = out_transform(hp)


def _elu(x):
    return jnp.where(x > 0.0, x, jnp.exp(x) - 1.0)


def _elu_log_softmax(x):
    y = _elu(x)
    mx = jnp.max(y, axis=1, keepdims=True)
    lse = jnp.log(jnp.sum(jnp.exp(y - mx), axis=1, keepdims=True))
    return y - mx - lse


def _run_prep(h, wcat, a1, a2, nheads):
    n, fin = h.shape
    fcat = wcat.shape[1]
    grid = (n // BR,)
    return pl.pallas_call(
        _prep_kernel,
        grid=grid,
        in_specs=[
            pl.BlockSpec((BR, fin), lambda i: (i, 0)),
            pl.BlockSpec((fin, fcat), lambda i: (0, 0)),
            pl.BlockSpec((fcat, nheads), lambda i: (0, 0)),
            pl.BlockSpec((fcat, nheads), lambda i: (0, 0)),
        ],
        out_specs=[
            pl.BlockSpec((BR, nheads * HW), lambda i: (i, 0)),
            pl.BlockSpec((BR, nheads), lambda i: (i, 0)),
            pl.BlockSpec((8, BR), lambda i: (0, i)),
            pl.BlockSpec((1, nheads), lambda i: (0, 0)),
        ],
        out_shape=[
            jax.ShapeDtypeStruct((n, nheads * HW), jnp.float32),
            jax.ShapeDtypeStruct((n, nheads), jnp.float32),
            jax.ShapeDtypeStruct((8, n), jnp.float32),
            jax.ShapeDtypeStruct((1, nheads), jnp.float32),
        ],
        scratch_shapes=[pltpu.VMEM((1, nheads), jnp.float32)],
        compiler_params=pltpu.CompilerParams(
            dimension_semantics=("arbitrary",)),
    )(h, wcat, a1, a2)


def _run_flash(adj, s1, s2t, s2max, wh, nheads, fout, out_transform):
    n = adj.shape[0]
    grid = (n // BR, n // BC)
    return pl.pallas_call(
        functools.partial(_flash_kernel, nheads, fout, out_transform),
        grid=grid,
        in_specs=[
            pl.BlockSpec((BR, BC), lambda i, j: (i, j)),
            pl.BlockSpec((BR, nheads), lambda i, j: (i, 0)),
            pl.BlockSpec((8, BC), lambda i, j: (0, j)),
            pl.BlockSpec((1, nheads), lambda i, j: (0, 0)),
            pl.BlockSpec((n, nheads * HW), lambda i, j: (0, 0)),
        ],
        out_specs=pl.BlockSpec((BR, nheads * fout), lambda i, j: (i, 0)),
        out_shape=jax.ShapeDtypeStruct((n, nheads * fout), jnp.float32),
        scratch_shapes=[
            pltpu.VMEM((BR, nheads * HW), jnp.float32),
            pltpu.VMEM((BR, 4 * nheads), jnp.float32),
        ],
        compiler_params=pltpu.CompilerParams(
            dimension_semantics=("parallel", "arbitrary")),
    )(adj, s1, s2t, s2max, wh)


def kernel(x, adj, W0, W1, W2, W3, a0, a1, a2, a3, Wout, aout):
    h = x.reshape(N, F_IN)
    adjm = adj.reshape(N, N)

    wcat = jnp.concatenate([W0, W1, W2, W3], axis=1)
    a_list = [a0, a1, a2, a3]
    eye = jnp.eye(NHEADS, dtype=jnp.float32)
    a1cat = jnp.concatenate(
        [a_list[k][:F_OUT] * eye[k] for k in range(NHEADS)], axis=0)
    a2cat = jnp.concatenate(
        [a_list[k][F_OUT:] * eye[k] for k in range(NHEADS)], axis=0)

    wh, s1, s2t, s2max = _run_prep(h, wcat, a1cat, a2cat, NHEADS)
    hc = _run_flash(adjm, s1, s2t, s2max, wh, NHEADS, F_OUT, _elu)

    who, s1o, s2to, s2maxo = _run_prep(hc, Wout, aout[:F_OUT], aout[F_OUT:], 1)
    out = _run_flash(adjm, s1o, s2to, s2maxo, who, 1, F_OUT, _elu_log_softmax)
    return out


# speedup vs baseline: 1.0054x; 1.0054x over previous
import functools

import jax
import jax.numpy as jnp
from jax.experimental import pallas as pl
from jax.experimental.pallas import tpu as pltpu

N = 4096
F_IN = 128
F_OUT = 64
NHEADS = 4
ALPHA = 0.2

BR = 256
BC = 1024
HW = 128


def _prep_kernel(h_ref, w_ref, a1_ref, a2_ref,
                 wh_ref, s1_ref, s2t_ref, s2max_ref, runmax_ref):
    i = pl.program_id(0)
    ni = pl.num_programs(0)
    nheads = s1_ref.shape[1]
    fout = w_ref.shape[1] // nheads

    wh = jnp.dot(h_ref[:], w_ref[:], preferred_element_type=jnp.float32)
    s1 = jnp.dot(wh, a1_ref[:], preferred_element_type=jnp.float32)
    s2 = jnp.dot(wh, a2_ref[:], preferred_element_type=jnp.float32)
    s1_ref[:] = s1

    br = wh.shape[0]
    ones = jnp.ones((br, 1), jnp.float32)
    zpad = jnp.zeros((br, HW - fout - 1), jnp.float32)
    parts = []
    for k in range(nheads):
        parts += [wh[:, k * fout:(k + 1) * fout], ones, zpad]
    wh_ref[:] = jnp.concatenate(parts, axis=1).astype(jnp.bfloat16)

    s2t = jnp.concatenate(
        [s2.T, jnp.zeros((8 - nheads, br), jnp.float32)], axis=0)
    s2t_ref[:] = s2t

    bmax = jnp.max(s2, axis=0, keepdims=True)

    @pl.when(i == 0)
    def _():
        runmax_ref[:] = jnp.full_like(runmax_ref, -jnp.inf)

    runmax_ref[:] = jnp.maximum(runmax_ref[:], bmax)

    @pl.when(i == ni - 1)
    def _():
        s2max_ref[:] = runmax_ref[:]


def _flash_kernel(nheads, fout, out_transform, emit_mask,
                  adj_ref, s1_ref, s2t_ref, s2max_ref, wh_ref, *refs):
    if emit_mask:
        out_ref, maskb_ref, acc_ref, row_ref = refs
    else:
        out_ref, acc_ref, row_ref = refs
    j = pl.program_id(1)
    nj = pl.num_programs(1)

    @pl.when(j == 0)
    def _init():
        acc_ref[:] = jnp.zeros_like(acc_ref)
        for k in range(nheads):
            s1c = s1_ref[:, k:k + 1]
            s2m = s2max_ref[0:1, k:k + 1]
            t = s1c + s2m
            mrow = jnp.maximum(jnp.where(t >= 0.0, t, ALPHA * t), 0.0)
            row = jnp.concatenate(
                [jnp.exp(t - mrow),
                 jnp.exp(ALPHA * t - mrow),
                 jnp.exp(-mrow),
                 -s1c], axis=1)
            row_ref[:, 4 * k:4 * k + 4] = row.astype(jnp.bfloat16)

    if emit_mask:
        mask = adj_ref[:].astype(jnp.bfloat16) > 0.0
        maskb_ref[:] = mask.astype(jnp.bfloat16)
    else:
        mask = adj_ref[:] > 0.0

    for k in range(nheads):
        s2r = s2t_ref[k:k + 1, :]
        s2m = s2max_ref[0:1, k:k + 1]
        e1 = row_ref[:, 4 * k:4 * k + 1]
        f1 = row_ref[:, 4 * k + 1:4 * k + 2]
        g = row_ref[:, 4 * k + 2:4 * k + 3]
        ns1 = row_ref[:, 4 * k + 3:4 * k + 4]
        e2 = jnp.exp(s2r - s2m).astype(jnp.bfloat16)
        f2 = jnp.exp(ALPHA * (s2r - s2m)).astype(jnp.bfloat16)
        s2rb = s2r.astype(jnp.bfloat16)

        c = s2rb >= ns1
        u = jnp.where(c, e2, f2)
        v = jnp.where(c, e1, f1)
        p = jnp.where(mask, u * v, g)
        whb = wh_ref[pl.ds(j * BC, BC), k * HW:(k + 1) * HW]
        acc_ref[:, k * HW:(k + 1) * HW] += jnp.dot(
            p, whb, preferred_element_type=jnp.float32)

    @pl.when(j == nj - 1)
    def _finish():
        for k in range(nheads):
            hp = (acc_ref[:, k * HW:k * HW + fout]
                  / acc_ref[:, k * HW + fout:k * HW + fout + 1])
            out_ref[:, k * fout:(k + 1) * fout] = out_transform(hp)


def _elu(x):
    return jnp.where(x > 0.0, x, jnp.exp(x) - 1.0)


def _elu_log_softmax(x):
    y = _elu(x)
    mx = jnp.max(y, axis=1, keepdims=True)
    lse = jnp.log(jnp.sum(jnp.exp(y - mx), axis=1, keepdims=True))
    return y - mx - lse


def _run_prep(h, wcat, a1, a2, nheads):
    n, fin = h.shape
    fcat = wcat.shape[1]
    grid = (n // BR,)
    return pl.pallas_call(
        _prep_kernel,
        grid=grid,
        in_specs=[
            pl.BlockSpec((BR, fin), lambda i: (i, 0)),
            pl.BlockSpec((fin, fcat), lambda i: (0, 0)),
            pl.BlockSpec((fcat, nheads), lambda i: (0, 0)),
            pl.BlockSpec((fcat, nheads), lambda i: (0, 0)),
        ],
        out_specs=[
            pl.BlockSpec((BR, nheads * HW), lambda i: (i, 0)),
            pl.BlockSpec((BR, nheads), lambda i: (i, 0)),
            pl.BlockSpec((8, BR), lambda i: (0, i)),
            pl.BlockSpec((1, nheads), lambda i: (0, 0)),
        ],
        out_shape=[
            jax.ShapeDtypeStruct((n, nheads * HW), jnp.bfloat16),
            jax.ShapeDtypeStruct((n, nheads), jnp.float32),
            jax.ShapeDtypeStruct((8, n), jnp.float32),
            jax.ShapeDtypeStruct((1, nheads), jnp.float32),
        ],
        scratch_shapes=[pltpu.VMEM((1, nheads), jnp.float32)],
        compiler_params=pltpu.CompilerParams(
            dimension_semantics=("arbitrary",)),
    )(h, wcat, a1, a2)


def _run_flash(adj, s1, s2t, s2max, wh, nheads, fout, out_transform,
               emit_mask):
    n = s1.shape[0]
    grid = (n // BR, n // BC)
    out_shape = [jax.ShapeDtypeStruct((n, nheads * fout), jnp.float32)]
    out_specs = [pl.BlockSpec((BR, nheads * fout), lambda i, j: (i, 0))]
    if emit_mask:
        out_shape.append(jax.ShapeDtypeStruct((n, n), jnp.bfloat16))
        out_specs.append(pl.BlockSpec((BR, BC), lambda i, j: (i, j)))
    res = pl.pallas_call(
        functools.partial(_flash_kernel, nheads, fout, out_transform,
                          emit_mask),
        grid=grid,
        in_specs=[
            pl.BlockSpec((BR, BC), lambda i, j: (i, j)),
            pl.BlockSpec((BR, nheads), lambda i, j: (i, 0)),
            pl.BlockSpec((8, BC), lambda i, j: (0, j)),
            pl.BlockSpec((1, nheads), lambda i, j: (0, 0)),
            pl.BlockSpec((n, nheads * HW), lambda i, j: (0, 0)),
        ],
        out_specs=out_specs,
        out_shape=out_shape,
        scratch_shapes=[
            pltpu.VMEM((BR, nheads * HW), jnp.float32),
            pltpu.VMEM((BR, 4 * nheads), jnp.bfloat16),
        ],
        compiler_params=pltpu.CompilerParams(
            dimension_semantics=("parallel", "arbitrary")),
    )(adj, s1, s2t, s2max, wh)
    return res


def kernel(x, adj, W0, W1, W2, W3, a0, a1, a2, a3, Wout, aout):
    h = x.reshape(N, F_IN)
    adjm = adj.reshape(N, N)

    wcat = jnp.concatenate([W0, W1, W2, W3], axis=1)
    a_list = [a0, a1, a2, a3]
    eye = jnp.eye(NHEADS, dtype=jnp.float32)
    a1cat = jnp.concatenate(
        [a_list[k][:F_OUT] * eye[k] for k in range(NHEADS)], axis=0)
    a2cat = jnp.concatenate(
        [a_list[k][F_OUT:] * eye[k] for k in range(NHEADS)], axis=0)

    wh, s1, s2t, s2max = _run_prep(h, wcat, a1cat, a2cat, NHEADS)
    hc, maskb = _run_flash(adjm, s1, s2t, s2max, wh, NHEADS, F_OUT,
                           _elu, emit_mask=True)

    who, s1o, s2to, s2maxo = _run_prep(hc, Wout, aout[:F_OUT], aout[F_OUT:], 1)
    (out,) = _run_flash(maskb, s1o, s2to, s2maxo, who, 1, F_OUT,
                        _elu_log_softmax, emit_mask=False)
    return out


# speedup vs baseline: 1.4818x; 1.4738x over previous
import functools

import jax
import jax.numpy as jnp
from jax.experimental import pallas as pl
from jax.experimental.pallas import tpu as pltpu

N = 4096
F_IN = 128
F_OUT = 64
NHEADS = 4
ALPHA = 0.2

BR = 256
BC = 4096
HW = 128


def _prep_kernel(h_ref, w_ref, a1_ref, a2_ref,
                 wh_ref, s1_ref, s2t_ref, s2max_ref, runmax_ref):
    i = pl.program_id(0)
    ni = pl.num_programs(0)
    nheads = s1_ref.shape[1]
    fout = w_ref.shape[1] // nheads

    wh = jnp.dot(h_ref[:], w_ref[:], preferred_element_type=jnp.float32)
    s1 = jnp.dot(wh, a1_ref[:], preferred_element_type=jnp.float32)
    s2 = jnp.dot(wh, a2_ref[:], preferred_element_type=jnp.float32)
    s1_ref[:] = s1

    br = wh.shape[0]
    ones = jnp.ones((br, 1), jnp.float32)
    zpad = jnp.zeros((br, HW - fout - 1), jnp.float32)
    parts = []
    for k in range(nheads):
        parts += [wh[:, k * fout:(k + 1) * fout], ones, zpad]
    wh_ref[:] = jnp.concatenate(parts, axis=1).astype(jnp.bfloat16)

    s2t = jnp.concatenate(
        [s2.T, jnp.zeros((8 - nheads, br), jnp.float32)], axis=0)
    s2t_ref[:] = s2t

    bmax = jnp.max(s2, axis=0, keepdims=True)

    @pl.when(i == 0)
    def _():
        runmax_ref[:] = jnp.full_like(runmax_ref, -jnp.inf)

    runmax_ref[:] = jnp.maximum(runmax_ref[:], bmax)

    @pl.when(i == ni - 1)
    def _():
        s2max_ref[:] = runmax_ref[:]


def _flash_kernel(nheads, fout, out_transform, emit_mask,
                  adj_ref, s1_ref, s2t_ref, s2max_ref, wh_ref, *refs):
    if emit_mask:
        out_ref, maskb_ref, acc_ref, row_ref = refs
    else:
        out_ref, acc_ref, row_ref = refs
    j = pl.program_id(1)
    nj = pl.num_programs(1)

    @pl.when(j == 0)
    def _init():
        acc_ref[:] = jnp.zeros_like(acc_ref)
        for k in range(nheads):
            s1c = s1_ref[:, k:k + 1]
            s2m = s2max_ref[0:1, k:k + 1]
            t = s1c + s2m
            mrow = jnp.maximum(jnp.where(t >= 0.0, t, ALPHA * t), 0.0)
            row = jnp.concatenate(
                [jnp.exp(t - mrow),
                 jnp.exp(ALPHA * t - mrow),
                 jnp.exp(-mrow),
                 -s1c], axis=1)
            row_ref[:, 4 * k:4 * k + 4] = row.astype(jnp.bfloat16)

    if emit_mask:
        mask = adj_ref[:].astype(jnp.bfloat16) > 0.0
        maskb_ref[:] = mask.astype(jnp.bfloat16)
    else:
        mask = adj_ref[:] > 0.0

    for k in range(nheads):
        s2r = s2t_ref[k:k + 1, :]
        s2m = s2max_ref[0:1, k:k + 1]
        e1 = row_ref[:, 4 * k:4 * k + 1]
        f1 = row_ref[:, 4 * k + 1:4 * k + 2]
        g = row_ref[:, 4 * k + 2:4 * k + 3]
        ns1 = row_ref[:, 4 * k + 3:4 * k + 4]
        e2 = jnp.exp(s2r - s2m).astype(jnp.bfloat16)
        f2 = jnp.exp(ALPHA * (s2r - s2m)).astype(jnp.bfloat16)
        s2rb = s2r.astype(jnp.bfloat16)

        c = s2rb >= ns1
        u = jnp.where(c, e2, f2)
        v = jnp.where(c, e1, f1)
        p = jnp.where(mask, u * v, g)
        whb = wh_ref[pl.ds(j * BC, BC), k * HW:(k + 1) * HW]
        acc_ref[:, k * HW:(k + 1) * HW] += jnp.dot(
            p, whb, preferred_element_type=jnp.float32)

    @pl.when(j == nj - 1)
    def _finish():
        for k in range(nheads):
            hp = (acc_ref[:, k * HW:k * HW + fout]
                  / acc_ref[:, k * HW + fout:k * HW + fout + 1])
            out_ref[:, k * fout:(k + 1) * fout] = out_transform(hp)


def _elu(x):
    return jnp.where(x > 0.0, x, jnp.exp(x) - 1.0)


def _elu_log_softmax(x):
    y = _elu(x)
    mx = jnp.max(y, axis=1, keepdims=True)
    lse = jnp.log(jnp.sum(jnp.exp(y - mx), axis=1, keepdims=True))
    return y - mx - lse


def _run_prep(h, wcat, a1, a2, nheads):
    n, fin = h.shape
    fcat = wcat.shape[1]
    grid = (n // BR,)
    return pl.pallas_call(
        _prep_kernel,
        grid=grid,
        in_specs=[
            pl.BlockSpec((BR, fin), lambda i: (i, 0)),
            pl.BlockSpec((fin, fcat), lambda i: (0, 0)),
            pl.BlockSpec((fcat, nheads), lambda i: (0, 0)),
            pl.BlockSpec((fcat, nheads), lambda i: (0, 0)),
        ],
        out_specs=[
            pl.BlockSpec((BR, nheads * HW), lambda i: (i, 0)),
            pl.BlockSpec((BR, nheads), lambda i: (i, 0)),
            pl.BlockSpec((8, BR), lambda i: (0, i)),
            pl.BlockSpec((1, nheads), lambda i: (0, 0)),
        ],
        out_shape=[
            jax.ShapeDtypeStruct((n, nheads * HW), jnp.bfloat16),
            jax.ShapeDtypeStruct((n, nheads), jnp.float32),
            jax.ShapeDtypeStruct((8, n), jnp.float32),
            jax.ShapeDtypeStruct((1, nheads), jnp.float32),
        ],
        scratch_shapes=[pltpu.VMEM((1, nheads), jnp.float32)],
        compiler_params=pltpu.CompilerParams(
            dimension_semantics=("arbitrary",)),
    )(h, wcat, a1, a2)


def _run_flash(adj, s1, s2t, s2max, wh, nheads, fout, out_transform,
               emit_mask):
    n = s1.shape[0]
    grid = (n // BR, n // BC)
    out_shape = [jax.ShapeDtypeStruct((n, nheads * fout), jnp.float32)]
    out_specs = [pl.BlockSpec((BR, nheads * fout), lambda i, j: (i, 0))]
    if emit_mask:
        out_shape.append(jax.ShapeDtypeStruct((n, n), jnp.bfloat16))
        out_specs.append(pl.BlockSpec((BR, BC), lambda i, j: (i, j)))
    res = pl.pallas_call(
        functools.partial(_flash_kernel, nheads, fout, out_transform,
                          emit_mask),
        grid=grid,
        in_specs=[
            pl.BlockSpec((BR, BC), lambda i, j: (i, j)),
            pl.BlockSpec((BR, nheads), lambda i, j: (i, 0)),
            pl.BlockSpec((8, BC), lambda i, j: (0, j)),
            pl.BlockSpec((1, nheads), lambda i, j: (0, 0)),
            pl.BlockSpec((n, nheads * HW), lambda i, j: (0, 0)),
        ],
        out_specs=out_specs,
        out_shape=out_shape,
        scratch_shapes=[
            pltpu.VMEM((BR, nheads * HW), jnp.float32),
            pltpu.VMEM((BR, 4 * nheads), jnp.bfloat16),
        ],
        compiler_params=pltpu.CompilerParams(
            dimension_semantics=("parallel", "arbitrary")),
    )(adj, s1, s2t, s2max, wh)
    return res


def kernel(x, adj, W0, W1, W2, W3, a0, a1, a2, a3, Wout, aout):
    h = x.reshape(N, F_IN)
    adjm = adj.reshape(N, N)

    wcat = jnp.concatenate([W0, W1, W2, W3], axis=1)
    a_list = [a0, a1, a2, a3]
    eye = jnp.eye(NHEADS, dtype=jnp.float32)
    a1cat = jnp.concatenate(
        [a_list[k][:F_OUT] * eye[k] for k in range(NHEADS)], axis=0)
    a2cat = jnp.concatenate(
        [a_list[k][F_OUT:] * eye[k] for k in range(NHEADS)], axis=0)

    wh, s1, s2t, s2max = _run_prep(h, wcat, a1cat, a2cat, NHEADS)
    hc, maskb = _run_flash(adjm, s1, s2t, s2max, wh, NHEADS, F_OUT,
                           _elu, emit_mask=True)

    who, s1o, s2to, s2maxo = _run_prep(hc, Wout, aout[:F_OUT], aout[F_OUT:], 1)
    (out,) = _run_flash(maskb, s1o, s2to, s2maxo, who, 1, F_OUT,
                        _elu_log_softmax, emit_mask=False)
    return out


# speedup vs baseline: 1.7128x; 1.1559x over previous
import functools

import jax
import jax.numpy as jnp
from jax.experimental import pallas as pl
from jax.experimental.pallas import tpu as pltpu

N = 4096
F_IN = 128
F_OUT = 64
NHEADS = 4
ALPHA = 0.2

BR = 512
BC = 4096
HW = 128


def _prep_kernel(h_ref, w_ref, a1_ref, a2_ref,
                 wh_ref, s1_ref, s2t_ref, s2max_ref, runmax_ref):
    i = pl.program_id(0)
    ni = pl.num_programs(0)
    nheads = s1_ref.shape[1]
    fout = w_ref.shape[1] // nheads

    wh = jnp.dot(h_ref[:], w_ref[:], preferred_element_type=jnp.float32)
    s1 = jnp.dot(wh, a1_ref[:], preferred_element_type=jnp.float32)
    s2 = jnp.dot(wh, a2_ref[:], preferred_element_type=jnp.float32)
    s1_ref[:] = s1

    br = wh.shape[0]
    ones = jnp.ones((br, 1), jnp.float32)
    zpad = jnp.zeros((br, HW - fout - 1), jnp.float32)
    parts = []
    for k in range(nheads):
        parts += [wh[:, k * fout:(k + 1) * fout], ones, zpad]
    wh_ref[:] = jnp.concatenate(parts, axis=1).astype(jnp.bfloat16)

    s2t = jnp.concatenate(
        [s2.T, jnp.zeros((8 - nheads, br), jnp.float32)], axis=0)
    s2t_ref[:] = s2t

    bmax = jnp.max(s2, axis=0, keepdims=True)

    @pl.when(i == 0)
    def _():
        runmax_ref[:] = jnp.full_like(runmax_ref, -jnp.inf)

    runmax_ref[:] = jnp.maximum(runmax_ref[:], bmax)

    @pl.when(i == ni - 1)
    def _():
        s2max_ref[:] = runmax_ref[:]


def _flash_kernel(nheads, fout, out_transform, emit_mask,
                  adj_ref, s1_ref, s2t_ref, s2max_ref, wh_ref, *refs):
    if emit_mask:
        out_ref, maskb_ref, acc_ref, row_ref = refs
    else:
        out_ref, acc_ref, row_ref = refs
    j = pl.program_id(1)
    nj = pl.num_programs(1)

    @pl.when(j == 0)
    def _init():
        acc_ref[:] = jnp.zeros_like(acc_ref)
        for k in range(nheads):
            s1c = s1_ref[:, k:k + 1]
            s2m = s2max_ref[0:1, k:k + 1]
            t = s1c + s2m
            mrow = jnp.maximum(jnp.where(t >= 0.0, t, ALPHA * t), 0.0)
            row = jnp.concatenate(
                [jnp.exp(t - mrow),
                 jnp.exp(ALPHA * t - mrow),
                 jnp.exp(-mrow),
                 -s1c], axis=1)
            row_ref[:, 4 * k:4 * k + 4] = row.astype(jnp.bfloat16)

    if emit_mask:
        mask = adj_ref[:].astype(jnp.bfloat16) > 0.0
        maskb_ref[:] = mask.astype(jnp.bfloat16)
    else:
        mask = adj_ref[:] > 0.0

    for k in range(nheads):
        s2r = s2t_ref[k:k + 1, :]
        s2m = s2max_ref[0:1, k:k + 1]
        e1 = row_ref[:, 4 * k:4 * k + 1]
        f1 = row_ref[:, 4 * k + 1:4 * k + 2]
        g = row_ref[:, 4 * k + 2:4 * k + 3]
        ns1 = row_ref[:, 4 * k + 3:4 * k + 4]
        e2 = jnp.exp(s2r - s2m).astype(jnp.bfloat16)
        f2 = jnp.exp(ALPHA * (s2r - s2m)).astype(jnp.bfloat16)
        s2rb = s2r.astype(jnp.bfloat16)

        c = s2rb >= ns1
        u = jnp.where(c, e2, f2)
        v = jnp.where(c, e1, f1)
        p = jnp.where(mask, u * v, g)
        whb = wh_ref[pl.ds(j * BC, BC), k * HW:(k + 1) * HW]
        acc_ref[:, k * HW:(k + 1) * HW] += jnp.dot(
            p, whb, preferred_element_type=jnp.float32)

    @pl.when(j == nj - 1)
    def _finish():
        for k in range(nheads):
            hp = (acc_ref[:, k * HW:k * HW + fout]
                  / acc_ref[:, k * HW + fout:k * HW + fout + 1])
            out_ref[:, k * fout:(k + 1) * fout] = out_transform(hp)


def _elu(x):
    return jnp.where(x > 0.0, x, jnp.exp(x) - 1.0)


def _elu_log_softmax(x):
    y = _elu(x)
    mx = jnp.max(y, axis=1, keepdims=True)
    lse = jnp.log(jnp.sum(jnp.exp(y - mx), axis=1, keepdims=True))
    return y - mx - lse


def _run_prep(h, wcat, a1, a2, nheads):
    n, fin = h.shape
    fcat = wcat.shape[1]
    grid = (n // BR,)
    return pl.pallas_call(
        _prep_kernel,
        grid=grid,
        in_specs=[
            pl.BlockSpec((BR, fin), lambda i: (i, 0)),
            pl.BlockSpec((fin, fcat), lambda i: (0, 0)),
            pl.BlockSpec((fcat, nheads), lambda i: (0, 0)),
            pl.BlockSpec((fcat, nheads), lambda i: (0, 0)),
        ],
        out_specs=[
            pl.BlockSpec((BR, nheads * HW), lambda i: (i, 0)),
            pl.BlockSpec((BR, nheads), lambda i: (i, 0)),
            pl.BlockSpec((8, BR), lambda i: (0, i)),
            pl.BlockSpec((1, nheads), lambda i: (0, 0)),
        ],
        out_shape=[
            jax.ShapeDtypeStruct((n, nheads * HW), jnp.bfloat16),
            jax.ShapeDtypeStruct((n, nheads), jnp.float32),
            jax.ShapeDtypeStruct((8, n), jnp.float32),
            jax.ShapeDtypeStruct((1, nheads), jnp.float32),
        ],
        scratch_shapes=[pltpu.VMEM((1, nheads), jnp.float32)],
        compiler_params=pltpu.CompilerParams(
            dimension_semantics=("arbitrary",)),
    )(h, wcat, a1, a2)


def _run_flash(adj, s1, s2t, s2max, wh, nheads, fout, out_transform,
               emit_mask):
    n = s1.shape[0]
    grid = (n // BR, n // BC)
    out_shape = [jax.ShapeDtypeStruct((n, nheads * fout), jnp.float32)]
    out_specs = [pl.BlockSpec((BR, nheads * fout), lambda i, j: (i, 0))]
    if emit_mask:
        out_shape.append(jax.ShapeDtypeStruct((n, n), jnp.bfloat16))
        out_specs.append(pl.BlockSpec((BR, BC), lambda i, j: (i, j)))
    res = pl.pallas_call(
        functools.partial(_flash_kernel, nheads, fout, out_transform,
                          emit_mask),
        grid=grid,
        in_specs=[
            pl.BlockSpec((BR, BC), lambda i, j: (i, j)),
            pl.BlockSpec((BR, nheads), lambda i, j: (i, 0)),
            pl.BlockSpec((8, BC), lambda i, j: (0, j)),
            pl.BlockSpec((1, nheads), lambda i, j: (0, 0)),
            pl.BlockSpec((n, nheads * HW), lambda i, j: (0, 0)),
        ],
        out_specs=out_specs,
        out_shape=out_shape,
        scratch_shapes=[
            pltpu.VMEM((BR, nheads * HW), jnp.float32),
            pltpu.VMEM((BR, 4 * nheads), jnp.bfloat16),
        ],
        compiler_params=pltpu.CompilerParams(
            dimension_semantics=("parallel", "arbitrary")),
    )(adj, s1, s2t, s2max, wh)
    return res


def kernel(x, adj, W0, W1, W2, W3, a0, a1, a2, a3, Wout, aout):
    h = x.reshape(N, F_IN)
    adjm = adj.reshape(N, N)

    wcat = jnp.concatenate([W0, W1, W2, W3], axis=1)
    a_list = [a0, a1, a2, a3]
    eye = jnp.eye(NHEADS, dtype=jnp.float32)
    a1cat = jnp.concatenate(
        [a_list[k][:F_OUT] * eye[k] for k in range(NHEADS)], axis=0)
    a2cat = jnp.concatenate(
        [a_list[k][F_OUT:] * eye[k] for k in range(NHEADS)], axis=0)

    wh, s1, s2t, s2max = _run_prep(h, wcat, a1cat, a2cat, NHEADS)
    hc, maskb = _run_flash(adjm, s1, s2t, s2max, wh, NHEADS, F_OUT,
                           _elu, emit_mask=True)

    who, s1o, s2to, s2maxo = _run_prep(hc, Wout, aout[:F_OUT], aout[F_OUT:], 1)
    (out,) = _run_flash(maskb, s1o, s2to, s2maxo, who, 1, F_OUT,
                        _elu_log_softmax, emit_mask=False)
    return out


# speedup vs baseline: 1.8973x; 1.1077x over previous
import jax
import jax.numpy as jnp
from jax.experimental import pallas as pl
from jax.experimental.pallas import tpu as pltpu

N = 4096
F_IN = 128
F_OUT = 64
NHEADS = 4
ALPHA = 0.2

BR = 512
BC = 4096
HW = 128


def _prep_kernel(h_ref, w_ref, a1_ref, a2_ref,
                 wh_ref, s1_ref, s2t_ref, s2max_ref, runmax_ref):
    i = pl.program_id(0)
    ni = pl.num_programs(0)
    nheads = s1_ref.shape[1]
    fout = w_ref.shape[1] // nheads

    wh = jnp.dot(h_ref[:], w_ref[:], preferred_element_type=jnp.float32)
    s1 = jnp.dot(wh, a1_ref[:], preferred_element_type=jnp.float32)
    s2 = jnp.dot(wh, a2_ref[:], preferred_element_type=jnp.float32)
    s1_ref[:] = s1

    br = wh.shape[0]
    ones = jnp.ones((br, 1), jnp.float32)
    zpad = jnp.zeros((br, HW - fout - 1), jnp.float32)
    parts = []
    for k in range(nheads):
        parts += [wh[:, k * fout:(k + 1) * fout], ones, zpad]
    wh_ref[:] = jnp.concatenate(parts, axis=1).astype(jnp.bfloat16)

    s2t = jnp.concatenate(
        [s2.T, jnp.zeros((8 - nheads, br), jnp.float32)], axis=0)
    s2t_ref[:] = s2t

    bmax = jnp.max(s2, axis=0, keepdims=True)

    @pl.when(i == 0)
    def _():
        runmax_ref[:] = jnp.full_like(runmax_ref, -jnp.inf)

    runmax_ref[:] = jnp.maximum(runmax_ref[:], bmax)

    @pl.when(i == ni - 1)
    def _():
        s2max_ref[:] = runmax_ref[:]


def _row_factors(s1c, s2m):
    t = s1c + s2m
    mrow = jnp.maximum(jnp.where(t >= 0.0, t, ALPHA * t), 0.0)
    row = jnp.concatenate(
        [jnp.exp(t - mrow),
         jnp.exp(ALPHA * t - mrow),
         jnp.exp(-mrow),
         -s1c], axis=1)
    return row.astype(jnp.bfloat16)


def _head_probs(mask, s2r, s2m, row, k):
    nh = row.shape[1] // 4
    e1 = row[:, k:k + 1]
    f1 = row[:, nh + k:nh + k + 1]
    g = row[:, 2 * nh + k:2 * nh + k + 1]
    ns1 = row[:, 3 * nh + k:3 * nh + k + 1]
    e2 = jnp.exp(s2r - s2m).astype(jnp.bfloat16)
    f2 = jnp.exp(ALPHA * (s2r - s2m)).astype(jnp.bfloat16)
    c = s2r.astype(jnp.bfloat16) >= ns1
    u = jnp.where(c, e2, f2)
    v = jnp.where(c, e1, f1)
    return jnp.where(mask, u * v, g)


def _elu(x):
    return jnp.where(x > 0.0, x, jnp.exp(x) - 1.0)


def _flash1_kernel(adj_ref, s1_ref, s2t_ref, s2max_ref, wh_ref,
                   wout_ref, a1o_ref, a2o_ref,
                   maskb_ref, who_ref, s1o_ref, s2to_ref, s2mo_ref,
                   runmax_ref):
    i = pl.program_id(0)
    ni = pl.num_programs(0)
    nheads = NHEADS
    fout = F_OUT

    row = _row_factors(s1_ref[:], s2max_ref[0:1, :])

    mask = adj_ref[:].astype(jnp.bfloat16) > 0.0
    maskb_ref[:] = mask.astype(jnp.bfloat16)

    hc_parts = []
    for k in range(nheads):
        p = _head_probs(mask, s2t_ref[k:k + 1, :], s2max_ref[0:1, k:k + 1],
                        row, k)
        acc = jnp.dot(p, wh_ref[:, k * HW:(k + 1) * HW],
                      preferred_element_type=jnp.float32)
        hp = acc[:, :fout] / acc[:, fout:fout + 1]
        hc_parts.append(_elu(hp))
    hcb = jnp.concatenate(hc_parts, axis=1)

    who = jnp.dot(hcb, wout_ref[:], preferred_element_type=jnp.float32)
    s1o = jnp.dot(who, a1o_ref[:], preferred_element_type=jnp.float32)
    s2o = jnp.dot(who, a2o_ref[:], preferred_element_type=jnp.float32)

    br = who.shape[0]
    who_ref[:] = jnp.concatenate(
        [who, jnp.ones((br, 1), jnp.float32),
         jnp.zeros((br, HW - fout - 1), jnp.float32)],
        axis=1).astype(jnp.bfloat16)
    s1o_ref[:] = s1o
    s2to_ref[:] = jnp.concatenate(
        [s2o.T, jnp.zeros((7, br), jnp.float32)], axis=0)

    @pl.when(i == 0)
    def _():
        runmax_ref[:] = jnp.full_like(runmax_ref, -jnp.inf)

    runmax_ref[:] = jnp.maximum(runmax_ref[:], jnp.max(s2o))

    @pl.when(i == ni - 1)
    def _():
        s2mo_ref[:] = runmax_ref[:]


def _flash2_kernel(maskb_ref, s1_ref, s2t_ref, s2max_ref, wh_ref, out_ref):
    fout = F_OUT
    row = _row_factors(s1_ref[:], s2max_ref[0:1, :])
    mask = maskb_ref[:] > 0.0
    p = _head_probs(mask, s2t_ref[0:1, :], s2max_ref[0:1, 0:1], row, 0)
    acc = jnp.dot(p, wh_ref[:, :HW], preferred_element_type=jnp.float32)
    hp = acc[:, :fout] / acc[:, fout:fout + 1]
    y = _elu(hp)
    mx = jnp.max(y, axis=1, keepdims=True)
    lse = jnp.log(jnp.sum(jnp.exp(y - mx), axis=1, keepdims=True))
    out_ref[:] = y - mx - lse


def _run_prep(h, wcat, a1, a2, nheads):
    n, fin = h.shape
    fcat = wcat.shape[1]
    grid = (n // BR,)
    return pl.pallas_call(
        _prep_kernel,
        grid=grid,
        in_specs=[
            pl.BlockSpec((BR, fin), lambda i: (i, 0)),
            pl.BlockSpec((fin, fcat), lambda i: (0, 0)),
            pl.BlockSpec((fcat, nheads), lambda i: (0, 0)),
            pl.BlockSpec((fcat, nheads), lambda i: (0, 0)),
        ],
        out_specs=[
            pl.BlockSpec((BR, nheads * HW), lambda i: (i, 0)),
            pl.BlockSpec((BR, nheads), lambda i: (i, 0)),
            pl.BlockSpec((8, BR), lambda i: (0, i)),
            pl.BlockSpec((1, nheads), lambda i: (0, 0)),
        ],
        out_shape=[
            jax.ShapeDtypeStruct((n, nheads * HW), jnp.bfloat16),
            jax.ShapeDtypeStruct((n, nheads), jnp.float32),
            jax.ShapeDtypeStruct((8, n), jnp.float32),
            jax.ShapeDtypeStruct((1, nheads), jnp.float32),
        ],
        scratch_shapes=[pltpu.VMEM((1, nheads), jnp.float32)],
        compiler_params=pltpu.CompilerParams(
            dimension_semantics=("arbitrary",)),
    )(h, wcat, a1, a2)


def kernel(x, adj, W0, W1, W2, W3, a0, a1, a2, a3, Wout, aout):
    h = x.reshape(N, F_IN)
    adjm = adj.reshape(N, N)

    wcat = jnp.concatenate([W0, W1, W2, W3], axis=1)
    a_list = [a0, a1, a2, a3]
    eye = jnp.eye(NHEADS, dtype=jnp.float32)
    a1cat = jnp.concatenate(
        [a_list[k][:F_OUT] * eye[k] for k in range(NHEADS)], axis=0)
    a2cat = jnp.concatenate(
        [a_list[k][F_OUT:] * eye[k] for k in range(NHEADS)], axis=0)

    wh, s1, s2t, s2max = _run_prep(h, wcat, a1cat, a2cat, NHEADS)

    grid = (N // BR, 1)
    maskb, who, s1o, s2to, s2mo = pl.pallas_call(
        _flash1_kernel,
        grid=grid,
        in_specs=[
            pl.BlockSpec((BR, BC), lambda i, j: (i, j)),
            pl.BlockSpec((BR, NHEADS), lambda i, j: (i, 0)),
            pl.BlockSpec((8, BC), lambda i, j: (0, j)),
            pl.BlockSpec((1, NHEADS), lambda i, j: (0, 0)),
            pl.BlockSpec((N, NHEADS * HW), lambda i, j: (0, 0)),
            pl.BlockSpec((NHEADS * F_OUT, F_OUT), lambda i, j: (0, 0)),
            pl.BlockSpec((F_OUT, 1), lambda i, j: (0, 0)),
            pl.BlockSpec((F_OUT, 1), lambda i, j: (0, 0)),
        ],
        out_specs=[
            pl.BlockSpec((BR, BC), lambda i, j: (i, j)),
            pl.BlockSpec((BR, HW), lambda i, j: (i, 0)),
            pl.BlockSpec((BR, 1), lambda i, j: (i, 0)),
            pl.BlockSpec((8, BR), lambda i, j: (0, i)),
            pl.BlockSpec((1, 1), lambda i, j: (0, 0)),
        ],
        out_shape=[
            jax.ShapeDtypeStruct((N, N), jnp.bfloat16),
            jax.ShapeDtypeStruct((N, HW), jnp.bfloat16),
            jax.ShapeDtypeStruct((N, 1), jnp.float32),
            jax.ShapeDtypeStruct((8, N), jnp.float32),
            jax.ShapeDtypeStruct((1, 1), jnp.float32),
        ],
        scratch_shapes=[pltpu.VMEM((1, 1), jnp.float32)],
        compiler_params=pltpu.CompilerParams(
            dimension_semantics=("arbitrary", "arbitrary")),
    )(adjm, s1, s2t, s2max, wh, Wout, aout[:F_OUT], aout[F_OUT:])

    out = pl.pallas_call(
        _flash2_kernel,
        grid=grid,
        in_specs=[
            pl.BlockSpec((BR, BC), lambda i, j: (i, j)),
            pl.BlockSpec((BR, 1), lambda i, j: (i, 0)),
            pl.BlockSpec((8, BC), lambda i, j: (0, j)),
            pl.BlockSpec((1, 1), lambda i, j: (0, 0)),
            pl.BlockSpec((N, HW), lambda i, j: (0, 0)),
        ],
        out_specs=pl.BlockSpec((BR, F_OUT), lambda i, j: (i, 0)),
        out_shape=jax.ShapeDtypeStruct((N, F_OUT), jnp.float32),
        compiler_params=pltpu.CompilerParams(
            dimension_semantics=("parallel", "arbitrary")),
    )(maskb, s1o, s2to, s2mo, who)
    return out


# speedup vs baseline: 2.0998x; 1.1067x over previous
import jax
import jax.numpy as jnp
from jax.experimental import pallas as pl
from jax.experimental.pallas import tpu as pltpu

N = 4096
F_IN = 128
F_OUT = 64
NHEADS = 4
ALPHA = 0.2

BR = 512
BC = 4096
HW = 128


def _prep_kernel(h_ref, w_ref, a1_ref, a2_ref,
                 wh_ref, s1_ref, s2t_ref, s2max_ref, runmax_ref):
    i = pl.program_id(0)
    ni = pl.num_programs(0)
    nheads = s1_ref.shape[1]
    fout = w_ref.shape[1] // nheads

    wh = jnp.dot(h_ref[:], w_ref[:], preferred_element_type=jnp.float32)
    s1 = jnp.dot(wh, a1_ref[:], preferred_element_type=jnp.float32)
    s2 = jnp.dot(wh, a2_ref[:], preferred_element_type=jnp.float32)
    s1_ref[:] = s1

    br = wh.shape[0]
    ones = jnp.ones((br, 1), jnp.float32)
    zpad = jnp.zeros((br, HW - fout - 1), jnp.float32)
    parts = []
    for k in range(nheads):
        parts += [wh[:, k * fout:(k + 1) * fout], ones, zpad]
    wh_ref[:] = jnp.concatenate(parts, axis=1).astype(jnp.bfloat16)

    s2t = jnp.concatenate(
        [s2.T, jnp.zeros((8 - nheads, br), jnp.float32)], axis=0)
    s2t_ref[:] = s2t

    bmax = jnp.max(s2, axis=0, keepdims=True)

    @pl.when(i == 0)
    def _():
        runmax_ref[:] = jnp.full_like(runmax_ref, -jnp.inf)

    runmax_ref[:] = jnp.maximum(runmax_ref[:], bmax)

    @pl.when(i == ni - 1)
    def _():
        s2max_ref[:] = runmax_ref[:]


def _row_factors(s1c, s2m):
    t = s1c + s2m
    mrow = jnp.maximum(jnp.where(t >= 0.0, t, ALPHA * t), 0.0)
    row = jnp.concatenate(
        [jnp.exp(t - mrow),
         jnp.exp(ALPHA * t - mrow),
         jnp.exp(-mrow),
         -s1c], axis=1)
    return row.astype(jnp.bfloat16)


def _head_probs(mask, s2r, s2m, row, k):
    nh = row.shape[1] // 4
    e1 = row[:, k:k + 1]
    f1 = row[:, nh + k:nh + k + 1]
    g = row[:, 2 * nh + k:2 * nh + k + 1]
    ns1 = row[:, 3 * nh + k:3 * nh + k + 1]
    e2 = jnp.exp(s2r - s2m).astype(jnp.bfloat16)
    f2 = jnp.exp(ALPHA * (s2r - s2m)).astype(jnp.bfloat16)
    c = s2r.astype(jnp.bfloat16) >= ns1
    u = jnp.where(c, e2, f2)
    v = jnp.where(c, e1, f1)
    return jnp.where(mask, u * v, g)


def _elu(x):
    return jnp.where(x > 0.0, x, jnp.exp(x) - 1.0)


def _flash1_kernel(adj_ref, s1_ref, s2t_ref, s2max_ref, wh_ref,
                   wout_ref, a1o_ref, a2o_ref,
                   maskb_ref, who_ref, s1o_ref, s2to_ref, s2mo_ref,
                   runmax_ref):
    i = pl.program_id(0)
    ni = pl.num_programs(0)
    nheads = NHEADS
    fout = F_OUT

    row = _row_factors(s1_ref[:], s2max_ref[0:1, :])

    mask = adj_ref[:].astype(jnp.bfloat16) > 0.0
    maskb_ref[:] = mask.astype(jnp.int8)

    hc_parts = []
    for k in range(nheads):
        p = _head_probs(mask, s2t_ref[k:k + 1, :], s2max_ref[0:1, k:k + 1],
                        row, k)
        acc = jnp.dot(p, wh_ref[:, k * HW:(k + 1) * HW],
                      preferred_element_type=jnp.float32)
        hp = acc[:, :fout] / acc[:, fout:fout + 1]
        hc_parts.append(_elu(hp))
    hcb = jnp.concatenate(hc_parts, axis=1)

    who = jnp.dot(hcb, wout_ref[:], preferred_element_type=jnp.float32)
    s1o = jnp.dot(who, a1o_ref[:], preferred_element_type=jnp.float32)
    s2o = jnp.dot(who, a2o_ref[:], preferred_element_type=jnp.float32)

    br = who.shape[0]
    who_ref[:] = jnp.concatenate(
        [who, jnp.ones((br, 1), jnp.float32),
         jnp.zeros((br, HW - fout - 1), jnp.float32)],
        axis=1).astype(jnp.bfloat16)
    s1o_ref[:] = s1o
    s2to_ref[:] = jnp.concatenate(
        [s2o.T, jnp.zeros((7, br), jnp.float32)], axis=0)

    @pl.when(i == 0)
    def _():
        runmax_ref[:] = jnp.full_like(runmax_ref, -jnp.inf)

    runmax_ref[:] = jnp.maximum(runmax_ref[:], jnp.max(s2o))

    @pl.when(i == ni - 1)
    def _():
        s2mo_ref[:] = runmax_ref[:]


def _flash2_kernel(maskb_ref, s1_ref, s2t_ref, s2max_ref, wh_ref, out_ref):
    fout = F_OUT
    row = _row_factors(s1_ref[:], s2max_ref[0:1, :])
    mask = maskb_ref[:].astype(jnp.bfloat16) > 0.0
    p = _head_probs(mask, s2t_ref[0:1, :], s2max_ref[0:1, 0:1], row, 0)
    acc = jnp.dot(p, wh_ref[:, :HW], preferred_element_type=jnp.float32)
    hp = acc[:, :fout] / acc[:, fout:fout + 1]
    y = _elu(hp)
    mx = jnp.max(y, axis=1, keepdims=True)
    lse = jnp.log(jnp.sum(jnp.exp(y - mx), axis=1, keepdims=True))
    out_ref[:] = y - mx - lse


def _run_prep(h, wcat, a1, a2, nheads):
    n, fin = h.shape
    fcat = wcat.shape[1]
    grid = (n // BR,)
    return pl.pallas_call(
        _prep_kernel,
        grid=grid,
        in_specs=[
            pl.BlockSpec((BR, fin), lambda i: (i, 0)),
            pl.BlockSpec((fin, fcat), lambda i: (0, 0)),
            pl.BlockSpec((fcat, nheads), lambda i: (0, 0)),
            pl.BlockSpec((fcat, nheads), lambda i: (0, 0)),
        ],
        out_specs=[
            pl.BlockSpec((BR, nheads * HW), lambda i: (i, 0)),
            pl.BlockSpec((BR, nheads), lambda i: (i, 0)),
            pl.BlockSpec((8, BR), lambda i: (0, i)),
            pl.BlockSpec((1, nheads), lambda i: (0, 0)),
        ],
        out_shape=[
            jax.ShapeDtypeStruct((n, nheads * HW), jnp.bfloat16),
            jax.ShapeDtypeStruct((n, nheads), jnp.float32),
            jax.ShapeDtypeStruct((8, n), jnp.float32),
            jax.ShapeDtypeStruct((1, nheads), jnp.float32),
        ],
        scratch_shapes=[pltpu.VMEM((1, nheads), jnp.float32)],
        compiler_params=pltpu.CompilerParams(
            dimension_semantics=("arbitrary",)),
    )(h, wcat, a1, a2)


def kernel(x, adj, W0, W1, W2, W3, a0, a1, a2, a3, Wout, aout):
    h = x.reshape(N, F_IN)
    adjm = adj.reshape(N, N)

    wcat = jnp.concatenate([W0, W1, W2, W3], axis=1)
    a_list = [a0, a1, a2, a3]
    eye = jnp.eye(NHEADS, dtype=jnp.float32)
    a1cat = jnp.concatenate(
        [a_list[k][:F_OUT] * eye[k] for k in range(NHEADS)], axis=0)
    a2cat = jnp.concatenate(
        [a_list[k][F_OUT:] * eye[k] for k in range(NHEADS)], axis=0)

    wh, s1, s2t, s2max = _run_prep(h, wcat, a1cat, a2cat, NHEADS)

    grid = (N // BR, 1)
    maskb, who, s1o, s2to, s2mo = pl.pallas_call(
        _flash1_kernel,
        grid=grid,
        in_specs=[
            pl.BlockSpec((BR, BC), lambda i, j: (i, j)),
            pl.BlockSpec((BR, NHEADS), lambda i, j: (i, 0)),
            pl.BlockSpec((8, BC), lambda i, j: (0, j)),
            pl.BlockSpec((1, NHEADS), lambda i, j: (0, 0)),
            pl.BlockSpec((N, NHEADS * HW), lambda i, j: (0, 0)),
            pl.BlockSpec((NHEADS * F_OUT, F_OUT), lambda i, j: (0, 0)),
            pl.BlockSpec((F_OUT, 1), lambda i, j: (0, 0)),
            pl.BlockSpec((F_OUT, 1), lambda i, j: (0, 0)),
        ],
        out_specs=[
            pl.BlockSpec((BR, BC), lambda i, j: (i, j)),
            pl.BlockSpec((BR, HW), lambda i, j: (i, 0)),
            pl.BlockSpec((BR, 1), lambda i, j: (i, 0)),
            pl.BlockSpec((8, BR), lambda i, j: (0, i)),
            pl.BlockSpec((1, 1), lambda i, j: (0, 0)),
        ],
        out_shape=[
            jax.ShapeDtypeStruct((N, N), jnp.int8),
            jax.ShapeDtypeStruct((N, HW), jnp.bfloat16),
            jax.ShapeDtypeStruct((N, 1), jnp.float32),
            jax.ShapeDtypeStruct((8, N), jnp.float32),
            jax.ShapeDtypeStruct((1, 1), jnp.float32),
        ],
        scratch_shapes=[pltpu.VMEM((1, 1), jnp.float32)],
        compiler_params=pltpu.CompilerParams(
            dimension_semantics=("arbitrary", "arbitrary")),
    )(adjm, s1, s2t, s2max, wh, Wout, aout[:F_OUT], aout[F_OUT:])

    out = pl.pallas_call(
        _flash2_kernel,
        grid=grid,
        in_specs=[
            pl.BlockSpec((BR, BC), lambda i, j: (i, j)),
            pl.BlockSpec((BR, 1), lambda i, j: (i, 0)),
            pl.BlockSpec((8, BC), lambda i, j: (0, j)),
            pl.BlockSpec((1, 1), lambda i, j: (0, 0)),
            pl.BlockSpec((N, HW), lambda i, j: (0, 0)),
        ],
        out_specs=pl.BlockSpec((BR, F_OUT), lambda i, j: (i, 0)),
        out_shape=jax.ShapeDtypeStruct((N, F_OUT), jnp.float32),
        compiler_params=pltpu.CompilerParams(
            dimension_semantics=("parallel", "arbitrary")),
    )(maskb, s1o, s2to, s2mo, who)
    return out


# speedup vs baseline: 2.1757x; 1.0362x over previous
import jax
import jax.numpy as jnp
from jax.experimental import pallas as pl
from jax.experimental.pallas import tpu as pltpu

N = 4096
F_IN = 128
F_OUT = 64
NHEADS = 4
ALPHA = 0.2

BR = 512
BC = 4096
NI = N // BR
HW = 128


def _prep_kernel(h_ref, w_ref, a1_ref, a2_ref,
                 wh_ref, s1_ref, s2t_ref, s2max_ref, runmax_ref):
    i = pl.program_id(0)
    ni = pl.num_programs(0)
    nheads = s1_ref.shape[1]
    fout = w_ref.shape[1] // nheads

    wh = jnp.dot(h_ref[:], w_ref[:], preferred_element_type=jnp.float32)
    s1 = jnp.dot(wh, a1_ref[:], preferred_element_type=jnp.float32)
    s2 = jnp.dot(wh, a2_ref[:], preferred_element_type=jnp.float32)
    s1_ref[:] = s1

    br = wh.shape[0]
    ones = jnp.ones((br, 1), jnp.float32)
    zpad = jnp.zeros((br, HW - fout - 1), jnp.float32)
    parts = []
    for k in range(nheads):
        parts += [wh[:, k * fout:(k + 1) * fout], ones, zpad]
    wh_ref[:] = jnp.concatenate(parts, axis=1).astype(jnp.bfloat16)

    s2t = jnp.concatenate(
        [s2.T, jnp.zeros((8 - nheads, br), jnp.float32)], axis=0)
    s2t_ref[:] = s2t

    bmax = jnp.max(s2, axis=0, keepdims=True)

    @pl.when(i == 0)
    def _():
        runmax_ref[:] = jnp.full_like(runmax_ref, -jnp.inf)

    runmax_ref[:] = jnp.maximum(runmax_ref[:], bmax)

    @pl.when(i == ni - 1)
    def _():
        s2max_ref[:] = runmax_ref[:]


def _row_factors(s1c, s2m):
    t = s1c + s2m
    mrow = jnp.maximum(jnp.where(t >= 0.0, t, ALPHA * t), 0.0)
    row = jnp.concatenate(
        [jnp.exp(t - mrow),
         jnp.exp(ALPHA * t - mrow),
         jnp.exp(-mrow),
         -s1c], axis=1)
    return row.astype(jnp.bfloat16)


def _head_probs(mask, s2r, s2m, row, k):
    nh = row.shape[1] // 4
    e1 = row[:, k:k + 1]
    f1 = row[:, nh + k:nh + k + 1]
    g = row[:, 2 * nh + k:2 * nh + k + 1]
    ns1 = row[:, 3 * nh + k:3 * nh + k + 1]
    e2 = jnp.exp(s2r - s2m).astype(jnp.bfloat16)
    f2 = jnp.exp(ALPHA * (s2r - s2m)).astype(jnp.bfloat16)
    c = s2r.astype(jnp.bfloat16) >= ns1
    u = jnp.where(c, e2, f2)
    v = jnp.where(c, e1, f1)
    return jnp.where(mask, u * v, g)


def _elu(x):
    return jnp.where(x > 0.0, x, jnp.exp(x) - 1.0)


def _fused_kernel(adj_ref, s1_ref, s2t_ref, s2max_ref, wh_ref,
                  wout_ref, a1o_ref, a2o_ref, out_ref,
                  mask_scr, who_scr, s1o_scr, s2to_scr, s2mo_scr):
    t = pl.program_id(0)
    i = pl.program_id(1)
    ni = pl.num_programs(1)
    fout = F_OUT

    @pl.when(t == 0)
    def _phase0():
        row = _row_factors(s1_ref[:], s2max_ref[0:1, :])
        mask = adj_ref[:].astype(jnp.bfloat16) > 0.0
        mask_scr[pl.ds(i * BR, BR), :] = mask.astype(jnp.int8)

        hc_parts = []
        for k in range(NHEADS):
            p = _head_probs(mask, s2t_ref[k:k + 1, :],
                            s2max_ref[0:1, k:k + 1], row, k)
            acc = jnp.dot(p, wh_ref[:, k * HW:(k + 1) * HW],
                          preferred_element_type=jnp.float32)
            hp = acc[:, :fout] / acc[:, fout:fout + 1]
            hc_parts.append(_elu(hp))
        hcb = jnp.concatenate(hc_parts, axis=1)

        who = jnp.dot(hcb, wout_ref[:], preferred_element_type=jnp.float32)
        s1o = jnp.dot(who, a1o_ref[:], preferred_element_type=jnp.float32)
        s2o = jnp.dot(who, a2o_ref[:], preferred_element_type=jnp.float32)

        br = who.shape[0]
        who_scr[pl.ds(i * BR, BR), :] = jnp.concatenate(
            [who, jnp.ones((br, 1), jnp.float32),
             jnp.zeros((br, HW - fout - 1), jnp.float32)],
            axis=1).astype(jnp.bfloat16)
        s1o_scr[pl.ds(i * BR, BR), :] = s1o
        s2to_scr[:, pl.ds(i * BR, BR)] = jnp.concatenate(
            [s2o.T, jnp.zeros((7, br), jnp.float32)], axis=0)

        @pl.when(i == 0)
        def _():
            s2mo_scr[:] = jnp.full_like(s2mo_scr, -jnp.inf)

        s2mo_scr[:] = jnp.maximum(s2mo_scr[:], jnp.max(s2o))

    @pl.when(t == 1)
    def _phase1():
        s2m = s2mo_scr[0:1, 0:1]
        row = _row_factors(s1o_scr[pl.ds(i * BR, BR), :], s2m)
        mask = mask_scr[pl.ds(i * BR, BR), :].astype(jnp.bfloat16) > 0.0
        p = _head_probs(mask, s2to_scr[0:1, :], s2m, row, 0)
        acc = jnp.dot(p, who_scr[:], preferred_element_type=jnp.float32)
        hp = acc[:, :fout] / acc[:, fout:fout + 1]
        y = _elu(hp)
        mx = jnp.max(y, axis=1, keepdims=True)
        lse = jnp.log(jnp.sum(jnp.exp(y - mx), axis=1, keepdims=True))
        out_ref[:] = y - mx - lse


def _run_prep(h, wcat, a1, a2, nheads):
    n, fin = h.shape
    fcat = wcat.shape[1]
    grid = (n // BR,)
    return pl.pallas_call(
        _prep_kernel,
        grid=grid,
        in_specs=[
            pl.BlockSpec((BR, fin), lambda i: (i, 0)),
            pl.BlockSpec((fin, fcat), lambda i: (0, 0)),
            pl.BlockSpec((fcat, nheads), lambda i: (0, 0)),
            pl.BlockSpec((fcat, nheads), lambda i: (0, 0)),
        ],
        out_specs=[
            pl.BlockSpec((BR, nheads * HW), lambda i: (i, 0)),
            pl.BlockSpec((BR, nheads), lambda i: (i, 0)),
            pl.BlockSpec((8, BR), lambda i: (0, i)),
            pl.BlockSpec((1, nheads), lambda i: (0, 0)),
        ],
        out_shape=[
            jax.ShapeDtypeStruct((n, nheads * HW), jnp.bfloat16),
            jax.ShapeDtypeStruct((n, nheads), jnp.float32),
            jax.ShapeDtypeStruct((8, n), jnp.float32),
            jax.ShapeDtypeStruct((1, nheads), jnp.float32),
        ],
        scratch_shapes=[pltpu.VMEM((1, nheads), jnp.float32)],
        compiler_params=pltpu.CompilerParams(
            dimension_semantics=("arbitrary",)),
    )(h, wcat, a1, a2)


def kernel(x, adj, W0, W1, W2, W3, a0, a1, a2, a3, Wout, aout):
    h = x.reshape(N, F_IN)
    adjm = adj.reshape(N, N)

    wcat = jnp.concatenate([W0, W1, W2, W3], axis=1)
    a_list = [a0, a1, a2, a3]
    eye = jnp.eye(NHEADS, dtype=jnp.float32)
    a1cat = jnp.concatenate(
        [a_list[k][:F_OUT] * eye[k] for k in range(NHEADS)], axis=0)
    a2cat = jnp.concatenate(
        [a_list[k][F_OUT:] * eye[k] for k in range(NHEADS)], axis=0)

    wh, s1, s2t, s2max = _run_prep(h, wcat, a1cat, a2cat, NHEADS)

    out = pl.pallas_call(
        _fused_kernel,
        grid=(2, NI),
        in_specs=[
            pl.BlockSpec((BR, BC),
                         lambda t, i: (jnp.where(t == 0, i, NI - 1), 0)),
            pl.BlockSpec((BR, NHEADS), lambda t, i: (i, 0)),
            pl.BlockSpec((8, BC), lambda t, i: (0, 0)),
            pl.BlockSpec((1, NHEADS), lambda t, i: (0, 0)),
            pl.BlockSpec((N, NHEADS * HW), lambda t, i: (0, 0)),
            pl.BlockSpec((NHEADS * F_OUT, F_OUT), lambda t, i: (0, 0)),
            pl.BlockSpec((F_OUT, 1), lambda t, i: (0, 0)),
            pl.BlockSpec((F_OUT, 1), lambda t, i: (0, 0)),
        ],
        out_specs=pl.BlockSpec((BR, F_OUT),
                               lambda t, i: (jnp.where(t == 1, i, 0), 0)),
        out_shape=jax.ShapeDtypeStruct((N, F_OUT), jnp.float32),
        scratch_shapes=[
            pltpu.VMEM((N, N), jnp.int8),
            pltpu.VMEM((N, HW), jnp.bfloat16),
            pltpu.VMEM((N, 1), jnp.float32),
            pltpu.VMEM((8, N), jnp.float32),
            pltpu.VMEM((1, 1), jnp.float32),
        ],
        compiler_params=pltpu.CompilerParams(
            dimension_semantics=("arbitrary", "arbitrary")),
    )(adjm, s1, s2t, s2max, wh, Wout, aout[:F_OUT], aout[F_OUT:])
    return out


# speedup vs baseline: 2.3002x; 1.0573x over previous
import jax
import jax.numpy as jnp
from jax.experimental import pallas as pl
from jax.experimental.pallas import tpu as pltpu

N = 4096
F_IN = 128
F_OUT = 64
NHEADS = 4
ALPHA = 0.2

BR = 512
BC = 4096
NI = N // BR
HW = 128


def _aug(wh, fout):
    br = wh.shape[0]
    nheads = wh.shape[1] // fout
    ones = jnp.ones((br, 1), jnp.float32)
    zpad = jnp.zeros((br, HW - fout - 1), jnp.float32)
    parts = []
    for k in range(nheads):
        parts += [wh[:, k * fout:(k + 1) * fout], ones, zpad]
    return jnp.concatenate(parts, axis=1).astype(jnp.bfloat16)


def _row_factors(s1c, s2m):
    t = s1c + s2m
    mrow = jnp.maximum(jnp.where(t >= 0.0, t, ALPHA * t), 0.0)
    row = jnp.concatenate(
        [jnp.exp(t - mrow),
         jnp.exp(ALPHA * t - mrow),
         jnp.exp(-mrow),
         -s1c], axis=1)
    return row.astype(jnp.bfloat16)


def _head_probs(mask, s2r, s2m, row, k):
    nh = row.shape[1] // 4
    e1 = row[:, k:k + 1]
    f1 = row[:, nh + k:nh + k + 1]
    g = row[:, 2 * nh + k:2 * nh + k + 1]
    ns1 = row[:, 3 * nh + k:3 * nh + k + 1]
    e2 = jnp.exp(s2r - s2m).astype(jnp.bfloat16)
    f2 = jnp.exp(ALPHA * (s2r - s2m)).astype(jnp.bfloat16)
    c = s2r.astype(jnp.bfloat16) >= ns1
    u = jnp.where(c, e2, f2)
    v = jnp.where(c, e1, f1)
    return jnp.where(mask, u * v, g)


def _elu(x):
    return jnp.where(x > 0.0, x, jnp.exp(x) - 1.0)


def _fused_kernel(h_ref, adj_ref, w_ref, a1_ref, a2_ref,
                  wout_ref, a1o_ref, a2o_ref, out_ref,
                  wh_scr, s1_scr, s2t_scr, s2max_scr,
                  mask_scr, who_scr, s1o_scr, s2to_scr, s2mo_scr):
    t = pl.program_id(0)
    i = pl.program_id(1)
    ni = pl.num_programs(1)
    fout = F_OUT
    rows = pl.ds(i * BR, BR)

    @pl.when(t == 0)
    def _prep():
        wh = jnp.dot(h_ref[rows, :], w_ref[:],
                     preferred_element_type=jnp.float32)
        s1 = jnp.dot(wh, a1_ref[:], preferred_element_type=jnp.float32)
        s2 = jnp.dot(wh, a2_ref[:], preferred_element_type=jnp.float32)
        wh_scr[rows, :] = _aug(wh, fout)
        s1_scr[rows, :] = s1
        s2t_scr[:, rows] = jnp.concatenate(
            [s2.T, jnp.zeros((8 - NHEADS, BR), jnp.float32)], axis=0)
        bmax = jnp.max(s2, axis=0, keepdims=True)

        @pl.when(i == 0)
        def _():
            s2max_scr[:] = jnp.full_like(s2max_scr, -jnp.inf)

        s2max_scr[:] = jnp.maximum(s2max_scr[:], bmax)

    @pl.when(t == 1)
    def _heads():
        row = _row_factors(s1_scr[rows, :], s2max_scr[0:1, :])
        mask = adj_ref[:].astype(jnp.bfloat16) > 0.0
        mask_scr[rows, :] = mask.astype(jnp.int8)

        hc_parts = []
        for k in range(NHEADS):
            p = _head_probs(mask, s2t_scr[k:k + 1, :],
                            s2max_scr[0:1, k:k + 1], row, k)
            acc = jnp.dot(p, wh_scr[:, k * HW:(k + 1) * HW],
                          preferred_element_type=jnp.float32)
            hp = acc[:, :fout] / acc[:, fout:fout + 1]
            hc_parts.append(_elu(hp))
        hcb = jnp.concatenate(hc_parts, axis=1)

        who = jnp.dot(hcb, wout_ref[:], preferred_element_type=jnp.float32)
        s1o = jnp.dot(who, a1o_ref[:], preferred_element_type=jnp.float32)
        s2o = jnp.dot(who, a2o_ref[:], preferred_element_type=jnp.float32)

        who_scr[rows, :] = _aug(who, fout)
        s1o_scr[rows, :] = s1o
        s2to_scr[:, rows] = jnp.concatenate(
            [s2o.T, jnp.zeros((7, BR), jnp.float32)], axis=0)

        @pl.when(i == 0)
        def _():
            s2mo_scr[:] = jnp.full_like(s2mo_scr, -jnp.inf)

        s2mo_scr[:] = jnp.maximum(s2mo_scr[:], jnp.max(s2o))

    @pl.when(t == 2)
    def _outhead():
        s2m = s2mo_scr[0:1, 0:1]
        row = _row_factors(s1o_scr[rows, :], s2m)
        mask = mask_scr[rows, :].astype(jnp.bfloat16) > 0.0
        p = _head_probs(mask, s2to_scr[0:1, :], s2m, row, 0)
        acc = jnp.dot(p, who_scr[:], preferred_element_type=jnp.float32)
        hp = acc[:, :fout] / acc[:, fout:fout + 1]
        y = _elu(hp)
        mx = jnp.max(y, axis=1, keepdims=True)
        lse = jnp.log(jnp.sum(jnp.exp(y - mx), axis=1, keepdims=True))
        out_ref[:] = y - mx - lse


def kernel(x, adj, W0, W1, W2, W3, a0, a1, a2, a3, Wout, aout):
    h = x.reshape(N, F_IN)
    adjm = adj.reshape(N, N)

    wcat = jnp.concatenate([W0, W1, W2, W3], axis=1)
    a_list = [a0, a1, a2, a3]
    eye = jnp.eye(NHEADS, dtype=jnp.float32)
    a1cat = jnp.concatenate(
        [a_list[k][:F_OUT] * eye[k] for k in range(NHEADS)], axis=0)
    a2cat = jnp.concatenate(
        [a_list[k][F_OUT:] * eye[k] for k in range(NHEADS)], axis=0)

    out = pl.pallas_call(
        _fused_kernel,
        grid=(3, NI),
        in_specs=[
            pl.BlockSpec((N, F_IN), lambda t, i: (0, 0)),
            pl.BlockSpec(
                (BR, BC),
                lambda t, i: (jnp.where(t == 1, i,
                                        jnp.where(t == 0, 0, NI - 1)), 0)),
            pl.BlockSpec((F_IN, NHEADS * F_OUT), lambda t, i: (0, 0)),
            pl.BlockSpec((NHEADS * F_OUT, NHEADS), lambda t, i: (0, 0)),
            pl.BlockSpec((NHEADS * F_OUT, NHEADS), lambda t, i: (0, 0)),
            pl.BlockSpec((NHEADS * F_OUT, F_OUT), lambda t, i: (0, 0)),
            pl.BlockSpec((F_OUT, 1), lambda t, i: (0, 0)),
            pl.BlockSpec((F_OUT, 1), lambda t, i: (0, 0)),
        ],
        out_specs=pl.BlockSpec((BR, F_OUT),
                               lambda t, i: (jnp.where(t == 2, i, 0), 0)),
        out_shape=jax.ShapeDtypeStruct((N, F_OUT), jnp.float32),
        scratch_shapes=[
            pltpu.VMEM((N, NHEADS * HW), jnp.bfloat16),
            pltpu.VMEM((N, NHEADS), jnp.float32),
            pltpu.VMEM((8, N), jnp.float32),
            pltpu.VMEM((1, NHEADS), jnp.float32),
            pltpu.VMEM((N, N), jnp.int8),
            pltpu.VMEM((N, HW), jnp.bfloat16),
            pltpu.VMEM((N, 1), jnp.float32),
            pltpu.VMEM((8, N), jnp.float32),
            pltpu.VMEM((1, 1), jnp.float32),
        ],
        compiler_params=pltpu.CompilerParams(
            dimension_semantics=("arbitrary", "arbitrary")),
    )(h, adjm, wcat, a1cat, a2cat, Wout, aout[:F_OUT], aout[F_OUT:])
    return out
